# Initial kernel scaffold; baseline (speedup 1.0000x reference)
import jax
import jax.numpy as jnp
from jax.experimental import pallas as pl


def kernel(params, laplacian_lambda_max, g_edge_index, user2item_edge_index, reverse_edge_index, item2user_edge_index, social_edge_index):
    raise NotImplementedError("write your pallas kernel here")


# speedup vs baseline: 1.6112x; 1.6112x over previous
import jax
import jax.numpy as jnp
from jax.experimental import pallas as pl
from jax.experimental.pallas import tpu as pltpu

N_PRED = 10000
N_ITEM = 10000
N = 20000
EMB = 128
E = 320000


def _lrelu(x, s):
    return jnp.where(x >= 0, x, s * x)


def _batch_norm(x, g, b, eps=1e-5):
    mu = x.mean(0, keepdims=True)
    var = ((x - mu) ** 2).mean(0, keepdims=True)
    return (x - mu) / jnp.sqrt(var + eps) * g + b


def _seg_softmax_z(s, seg, n):
    e = jnp.exp(s)
    z = jax.ops.segment_sum(e, seg, n)
    return e / (z[seg] + 1e-9)


def _gatv2(x, ei, p, n):
    src, dst = ei[0], ei[1]
    el = x @ p['Ws'] + p['bs']
    er = x @ p['Wd'] + p['bd']
    e = _lrelu(el[src] + er[dst], 0.2)
    sc = e @ p['a']
    al = _seg_softmax_z(sc, dst, n)
    out = jax.ops.segment_sum(al[:, None] * el[src], dst, n)
    return _lrelu(out, 0.01)


def _cheb(x, ei, W, b, n, lam):
    src, dst = ei[0], ei[1]
    deg = jax.ops.segment_sum(jnp.ones(src.shape[0], x.dtype), dst, n)
    dinv = jnp.where(deg > 0, 1.0 / jnp.sqrt(jnp.maximum(deg, 1e-9)), 0.0)
    def lhat(h):
        agg = jax.ops.segment_sum(h[src] * dinv[src][:, None], dst, n) * dinv[:, None]
        Lh = h - agg
        return (2.0 / lam) * Lh - h
    X0 = x
    out = X0 @ W[0]
    X1 = lhat(X0)
    out = out + X1 @ W[1]
    Xp, Xc = X0, X1
    for k in range(2, W.shape[0]):
        Xn = 2.0 * lhat(Xc) - Xp
        out = out + Xn @ W[k]
        Xp, Xc = Xc, Xn
    return _lrelu(out + b, 0.01)


def _mlp_bn(x, p):
    return _lrelu(_batch_norm(x @ p['W'] + p['b'], p['g'], p['be']), 0.01)


_BM = 2000


def _edge_dot_body(a_ref, b_ref, o_ref):
    o_ref[:, :] = jnp.sum(a_ref[:, :] * b_ref[:, :], axis=1, keepdims=True)


def _edge_dot(a, b):
    grid = (E // _BM,)
    return pl.pallas_call(
        _edge_dot_body,
        grid=grid,
        in_specs=[
            pl.BlockSpec((_BM, EMB), lambda i: (i, 0)),
            pl.BlockSpec((_BM, EMB), lambda i: (i, 0)),
        ],
        out_specs=pl.BlockSpec((_BM, 1), lambda i: (i, 0)),
        out_shape=jax.ShapeDtypeStruct((E, 1), jnp.float32),
    )(a, b)


def kernel(params, laplacian_lambda_max, g_edge_index, user2item_edge_index,
           reverse_edge_index, item2user_edge_index, social_edge_index):
    lam = laplacian_lambda_max[0]
    emb = _batch_norm(params['emb'], params['bn_g'], params['bn_b'])
    ui = emb
    h = _gatv2(ui, user2item_edge_index, params['gat_u2i'], N)
    ii = _gatv2(h, reverse_edge_index, params['gat_ii'], N)
    i2u = _gatv2(ui, item2user_edge_index, params['gat_i2u'], N)
    mask = jnp.sum(i2u, axis=1) != 0
    soc_emb = jnp.where(mask[:, None], i2u, ui)
    si = _gatv2(soc_emb, social_edge_index, params['gat_si'], N)
    spatial = _mlp_bn(jnp.concatenate([ii, si], axis=1), params['spatial_out'])
    hs = _cheb(ui, social_edge_index, params['cheb_W'], params['cheb_b'], N, lam)
    hs = _cheb(hs, social_edge_index, params['cheb_W'], params['cheb_b'], N, lam)
    social_pref = _gatv2(hs, social_edge_index, params['gat_spec'], N)
    h_uP = _mlp_bn(jnp.concatenate([spatial, ui], axis=1), params['mut_c'])
    h_uS = _mlp_bn(jnp.concatenate([social_pref, ui], axis=1), params['mut_s'])
    h_m = h_uP * h_uS
    h_mP = jnp.concatenate([h_m * jax.nn.softmax(h_uP, axis=1), h_uP], axis=1)
    h_mS = jnp.concatenate([h_m * jax.nn.softmax(h_uS, axis=1), h_uS], axis=1)
    h_new_P = _mlp_bn(h_mP, params['pred_p'])
    h_new_S = _mlp_bn(h_mS, params['pred_s'])
    new_ft = h_new_P + h_new_S
    raw_ft = _mlp_bn(emb, params['raw'])
    src, dst = g_edge_index[0], g_edge_index[1]
    return _edge_dot(new_ft[src], raw_ft[dst])


# speedup vs baseline: 3.1487x; 1.9543x over previous
import functools

import jax
import jax.numpy as jnp
from jax import lax
from jax.experimental import pallas as pl
from jax.experimental.pallas import tpu as pltpu
from jax.experimental.pallas import tpu_sc as plsc

N_PRED = 10000
N_ITEM = 10000
N = 20000
EMB = 128
E = 320000

NC = 2
NS = 16
L = 16

NPAD = 20480
ZSTRIPE = NPAD // NS
HALF = N // NC
ACC_ROWS = 10112
ACC_STRIPE = ACC_ROWS // NS

C = 400
GRP = C // L
CW = 160
GRPW = CW // L
PT_ALL = E // (NC * NS)
PT_SC = E // NS

_mesh = plsc.VectorSubcoreMesh(core_axis_name="c", subcore_axis_name="s")
_f32 = jnp.float32
_i32 = jnp.int32


_GDN = lax.GatherDimensionNumbers(offset_dims=(), collapsed_slice_dims=(0,),
                                  start_index_map=(0,))


def _perm16(x, idx):
    return lax.gather(x, idx[:, None], _GDN, slice_sizes=(1,),
                      mode=lax.GatherScatterMode.PROMISE_IN_BOUNDS)


def _hsum_all(x):
    lane = lax.iota(_i32, L)
    for step in (8, 4, 2, 1):
        x = x + _perm16(x, lane ^ step)
    return x


def _zero_vec(ref, n):
    def body(i, _):
        ref[pl.ds(i * L, L)] = jnp.zeros((L,), _f32)
        return 0
    lax.fori_loop(0, n // L, body, 0)


def _zero_rows(ref, rows):
    def body(i, _):
        for j in range(EMB // L):
            ref[i, pl.ds(j * L, L)] = jnp.zeros((L,), _f32)
        return 0
    lax.fori_loop(0, rows, body, 0)


def _gat_scores_body(el_h, er_h, a_h, src_h, dst_h, ee_h, zp_h,
                     idxs_v, idxd_v, ee_v, Lb, Rb, a_v, z_sh, sem1, sem2):
    c = lax.axis_index("c")
    s = lax.axis_index("s")
    wid = s * NC + c

    _zero_vec(ee_v, C)
    for off in (0, 400, 800):
        pltpu.sync_copy(ee_v, z_sh.at[pl.ds(s * ZSTRIPE + off, 400)])
    pltpu.sync_copy(ee_v.at[pl.ds(0, 80)], z_sh.at[pl.ds(s * ZSTRIPE + 1200, 80)])
    plsc.subcore_barrier()

    pltpu.sync_copy(a_h, a_v)

    def chunk(i, _):
        base = wid * PT_ALL + i * C
        pltpu.sync_copy(src_h.at[pl.ds(base, C)], idxs_v)
        pltpu.sync_copy(dst_h.at[pl.ds(base, C)], idxd_v)
        cp1 = pltpu.async_copy(el_h.at[idxs_v], Lb, sem1)
        cp2 = pltpu.async_copy(er_h.at[idxd_v], Rb, sem2)
        cp1.wait()
        cp2.wait()

        def group(g, _):
            a_vecs = [a_v[pl.ds(j * L, L)] for j in range(EMB // L)]
            lane = jax.lax.iota(_i32, L)
            sc16 = jnp.zeros((L,), _f32)
            for k in range(L):
                row = g * L + k
                acc_l = jnp.zeros((L,), _f32)
                acc_a = jnp.zeros((L,), _f32)
                for j in range(EMB // L):
                    sv = Lb[row, pl.ds(j * L, L)] + Rb[row, pl.ds(j * L, L)]
                    acc_l = acc_l + a_vecs[j] * sv
                    acc_a = acc_a + a_vecs[j] * jnp.abs(sv)
                sk = _hsum_all(0.6 * acc_l + 0.4 * acc_a)
                sc16 = jnp.where(lane == k, sk, sc16)
            ee_v[pl.ds(g * L, L)] = jnp.exp(sc16)
            return 0

        lax.fori_loop(0, GRP, group, 0)
        pltpu.sync_copy(ee_v, ee_h.at[pl.ds(base, C)])
        pltpu.sync_copy(ee_v, z_sh.at[idxd_v], add=True)
        return 0

    lax.fori_loop(0, PT_ALL // C, chunk, 0)
    plsc.subcore_barrier()
    pltpu.sync_copy(z_sh.at[pl.ds(s * ZSTRIPE, ZSTRIPE)],
                    zp_h.at[pl.ds(c * NPAD + s * ZSTRIPE, ZSTRIPE)])


def _sc_gat_scores(el, er, a, src, dst):
    fn = pl.kernel(
        _gat_scores_body,
        out_type=(jax.ShapeDtypeStruct((E,), _f32),
                  jax.ShapeDtypeStruct((NC * NPAD,), _f32)),
        scratch_types=[
            pltpu.VMEM((C,), _i32),
            pltpu.VMEM((C,), _i32),
            pltpu.VMEM((C,), _f32),
            pltpu.VMEM((C, EMB), _f32),
            pltpu.VMEM((C, EMB), _f32),
            pltpu.VMEM((EMB,), _f32),
            pltpu.VMEM_SHARED((NPAD,), _f32),
            pltpu.SemaphoreType.DMA,
            pltpu.SemaphoreType.DMA,
        ],
        mesh=_mesh,
    )
    return fn(el, er, a, src, dst)


def _deg_body(dst_h, zp_h, idxd_v, ones_v, z_sh, sem1):
    c = lax.axis_index("c")
    s = lax.axis_index("s")
    wid = s * NC + c

    _zero_vec(ones_v, C)
    for off in (0, 400, 800):
        pltpu.sync_copy(ones_v, z_sh.at[pl.ds(s * ZSTRIPE + off, 400)])
    pltpu.sync_copy(ones_v.at[pl.ds(0, 80)], z_sh.at[pl.ds(s * ZSTRIPE + 1200, 80)])
    plsc.subcore_barrier()

    def fill(i, _):
        ones_v[pl.ds(i * L, L)] = jnp.ones((L,), _f32)
        return 0
    lax.fori_loop(0, C // L, fill, 0)

    def chunk(i, _):
        base = wid * PT_ALL + i * C
        pltpu.sync_copy(dst_h.at[pl.ds(base, C)], idxd_v)
        pltpu.sync_copy(ones_v, z_sh.at[idxd_v], add=True)
        return 0

    lax.fori_loop(0, PT_ALL // C, chunk, 0)
    plsc.subcore_barrier()
    pltpu.sync_copy(z_sh.at[pl.ds(s * ZSTRIPE, ZSTRIPE)],
                    zp_h.at[pl.ds(c * NPAD + s * ZSTRIPE, ZSTRIPE)])


def _sc_deg(dst):
    fn = pl.kernel(
        _deg_body,
        out_type=jax.ShapeDtypeStruct((NC * NPAD,), _f32),
        scratch_types=[
            pltpu.VMEM((C,), _i32),
            pltpu.VMEM((C,), _f32),
            pltpu.VMEM_SHARED((NPAD,), _f32),
            pltpu.SemaphoreType.DMA,
        ],
        mesh=_mesh,
    )
    return fn(dst)


def _wscatter_body(weighted, rows_h, w_h, src_h, dst_h, out_h,
                   idxs_v, idxd_v, idxloc_v, w_v, Vb, acc_sh, sem1):
    c = lax.axis_index("c")
    s = lax.axis_index("s")
    half0 = c * HALF

    _zero_rows(Vb, CW)
    for off in (0, 160, 320, 480):
        ln = min(CW, ACC_STRIPE - off)
        pltpu.sync_copy(Vb.at[pl.ds(0, ln)],
                        acc_sh.at[pl.ds(s * ACC_STRIPE + off, ln)])
    plsc.subcore_barrier()

    def chunk(i, _):
        base = s * PT_SC + i * CW
        pltpu.sync_copy(src_h.at[pl.ds(base, CW)], idxs_v)
        pltpu.sync_copy(dst_h.at[pl.ds(base, CW)], idxd_v)
        if weighted:
            pltpu.sync_copy(w_h.at[pl.ds(base, CW)], w_v)
        pltpu.async_copy(rows_h.at[idxs_v], Vb, sem1).wait()

        def group(g, _):
            d16 = idxd_v[pl.ds(g * L, L)]
            inhalf = (d16 >= half0) & (d16 < half0 + HALF)
            loc = jnp.where(inhalf, d16 - half0, HALF)
            idxloc_v[pl.ds(g * L, L)] = loc
            if weighted:
                wv = w_v[pl.ds(g * L, L)]
                for k in range(L):
                    row = g * L + k
                    wk = wv[k]
                    for j in range(EMB // L):
                        Vb[row, pl.ds(j * L, L)] = Vb[row, pl.ds(j * L, L)] * wk
            return 0

        lax.fori_loop(0, GRPW, group, 0)
        pltpu.sync_copy(Vb, acc_sh.at[idxloc_v], add=True)
        return 0

    lax.fori_loop(0, PT_SC // CW, chunk, 0)
    plsc.subcore_barrier()

    @pl.when(s < NS - 1)
    def _():
        pltpu.sync_copy(acc_sh.at[pl.ds(s * ACC_STRIPE, ACC_STRIPE)],
                        out_h.at[pl.ds(c * HALF + s * ACC_STRIPE, ACC_STRIPE)])

    @pl.when(s == NS - 1)
    def _():
        pltpu.sync_copy(acc_sh.at[pl.ds((NS - 1) * ACC_STRIPE, HALF - (NS - 1) * ACC_STRIPE)],
                        out_h.at[pl.ds(c * HALF + (NS - 1) * ACC_STRIPE, HALF - (NS - 1) * ACC_STRIPE)])


def _sc_wscatter(rows, w, src, dst, weighted):
    fn = pl.kernel(
        functools.partial(_wscatter_body, weighted),
        out_type=jax.ShapeDtypeStruct((N, EMB), _f32),
        scratch_types=[
            pltpu.VMEM((CW,), _i32),
            pltpu.VMEM((CW,), _i32),
            pltpu.VMEM((CW,), _i32),
            pltpu.VMEM((CW,), _f32),
            pltpu.VMEM((CW, EMB), _f32),
            pltpu.VMEM_SHARED((ACC_ROWS, EMB), _f32),
            pltpu.SemaphoreType.DMA,
        ],
        mesh=_mesh,
    )
    return fn(rows, w, src, dst)


def _edge_dot_body(a_h, b_h, src_h, dst_h, out_h,
                   idxs_v, idxd_v, out_v, Ab, Bb, sem1, sem2):
    c = lax.axis_index("c")
    s = lax.axis_index("s")
    wid = s * NC + c

    def chunk(i, _):
        base = wid * PT_ALL + i * C
        pltpu.sync_copy(src_h.at[pl.ds(base, C)], idxs_v)
        pltpu.sync_copy(dst_h.at[pl.ds(base, C)], idxd_v)
        cp1 = pltpu.async_copy(a_h.at[idxs_v], Ab, sem1)
        cp2 = pltpu.async_copy(b_h.at[idxd_v], Bb, sem2)
        cp1.wait()
        cp2.wait()

        def group(g, _):
            lane = jax.lax.iota(_i32, L)
            dot16 = jnp.zeros((L,), _f32)
            for k in range(L):
                row = g * L + k
                acc = jnp.zeros((L,), _f32)
                for j in range(EMB // L):
                    acc = acc + Ab[row, pl.ds(j * L, L)] * Bb[row, pl.ds(j * L, L)]
                dot16 = jnp.where(lane == k, _hsum_all(acc), dot16)
            out_v[pl.ds(g * L, L)] = dot16
            return 0

        lax.fori_loop(0, GRP, group, 0)
        pltpu.sync_copy(out_v, out_h.at[pl.ds(base, C)])
        return 0

    lax.fori_loop(0, PT_ALL // C, chunk, 0)


def _sc_edge_dot(a, b, src, dst):
    fn = pl.kernel(
        _edge_dot_body,
        out_type=jax.ShapeDtypeStruct((E,), _f32),
        scratch_types=[
            pltpu.VMEM((C,), _i32),
            pltpu.VMEM((C,), _i32),
            pltpu.VMEM((C,), _f32),
            pltpu.VMEM((C, EMB), _f32),
            pltpu.VMEM((C, EMB), _f32),
            pltpu.SemaphoreType.DMA,
            pltpu.SemaphoreType.DMA,
        ],
        mesh=_mesh,
    )
    return fn(a, b, src, dst)


def _lrelu(x, s):
    return jnp.where(x >= 0, x, s * x)


def _batch_norm(x, g, b, eps=1e-5):
    mu = x.mean(0, keepdims=True)
    var = ((x - mu) ** 2).mean(0, keepdims=True)
    return (x - mu) / jnp.sqrt(var + eps) * g + b


def _gatv2(x, src, dst, p):
    el = x @ p['Ws'] + p['bs']
    er = x @ p['Wd'] + p['bd']
    ee, zpart = _sc_gat_scores(el, er, p['a'], src, dst)
    z = zpart[:N] + zpart[NPAD:NPAD + N]
    zinv = 1.0 / (z + 1e-9)
    w = ee * zinv[dst]
    out = _sc_wscatter(el, w, src, dst, weighted=True)
    return _lrelu(out, 0.01)


def _cheb(x, src, dst, W, b, lam, dinv):
    dcol = dinv[:, None]

    zw = jnp.zeros((E,), _f32)

    def lhat(h):
        agg = _sc_wscatter(h * dcol, zw, src, dst, weighted=False)
        return (2.0 / lam) * (h - agg * dcol) - h

    X0 = x
    X1 = lhat(X0)
    X2 = 2.0 * lhat(X1) - X0
    out = X0 @ W[0] + X1 @ W[1] + X2 @ W[2]
    return _lrelu(out + b, 0.01)


def _mlp_bn(x, p):
    return _lrelu(_batch_norm(x @ p['W'] + p['b'], p['g'], p['be']), 0.01)


def kernel(params, laplacian_lambda_max, g_edge_index, user2item_edge_index,
           reverse_edge_index, item2user_edge_index, social_edge_index):
    lam = laplacian_lambda_max[0]
    emb = _batch_norm(params['emb'], params['bn_g'], params['bn_b'])
    ui = emb

    u2i = user2item_edge_index
    rev = reverse_edge_index
    i2u_ei = item2user_edge_index
    soc = social_edge_index

    h = _gatv2(ui, u2i[0], u2i[1], params['gat_u2i'])
    ii = _gatv2(h, rev[0], rev[1], params['gat_ii'])
    i2u = _gatv2(ui, i2u_ei[0], i2u_ei[1], params['gat_i2u'])
    mask = jnp.sum(i2u, axis=1) != 0
    soc_emb = jnp.where(mask[:, None], i2u, ui)
    si = _gatv2(soc_emb, soc[0], soc[1], params['gat_si'])
    spatial = _mlp_bn(jnp.concatenate([ii, si], axis=1), params['spatial_out'])

    degpart = _sc_deg(soc[1])
    deg = degpart[:N] + degpart[NPAD:NPAD + N]
    dinv = jnp.where(deg > 0, 1.0 / jnp.sqrt(jnp.maximum(deg, 1e-9)), 0.0)
    hs = _cheb(ui, soc[0], soc[1], params['cheb_W'], params['cheb_b'], lam, dinv)
    hs = _cheb(hs, soc[0], soc[1], params['cheb_W'], params['cheb_b'], lam, dinv)
    social_pref = _gatv2(hs, soc[0], soc[1], params['gat_spec'])

    h_uP = _mlp_bn(jnp.concatenate([spatial, ui], axis=1), params['mut_c'])
    h_uS = _mlp_bn(jnp.concatenate([social_pref, ui], axis=1), params['mut_s'])
    h_m = h_uP * h_uS
    h_mP = jnp.concatenate([h_m * jax.nn.softmax(h_uP, axis=1), h_uP], axis=1)
    h_mS = jnp.concatenate([h_m * jax.nn.softmax(h_uS, axis=1), h_uS], axis=1)
    h_new_P = _mlp_bn(h_mP, params['pred_p'])
    h_new_S = _mlp_bn(h_mS, params['pred_s'])
    new_ft = h_new_P + h_new_S
    raw_ft = _mlp_bn(emb, params['raw'])

    ed = _sc_edge_dot(new_ft, raw_ft, g_edge_index[0], g_edge_index[1])
    return ed.reshape(E, 1)


# speedup vs baseline: 5.5094x; 1.7498x over previous
import functools

import jax
import jax.numpy as jnp
from jax import lax
from jax.experimental import pallas as pl
from jax.experimental.pallas import tpu as pltpu
from jax.experimental.pallas import tpu_sc as plsc

N_PRED = 10000
N_ITEM = 10000
N = 20000
EMB = 128
E = 320000

NC = 2
NS = 16
L = 16

NPAD = 20480
ZSTRIPE = NPAD // NS
HALF = N // NC
ACC_ROWS = 10112
ACC_STRIPE = ACC_ROWS // NS

C = 400
GRP = C // L
CW = 160
GRPW = CW // L
PT_ALL = E // (NC * NS)
PT_SC = E // NS

_mesh = plsc.VectorSubcoreMesh(core_axis_name="c", subcore_axis_name="s")
_f32 = jnp.float32
_i32 = jnp.int32


_GDN = lax.GatherDimensionNumbers(offset_dims=(), collapsed_slice_dims=(0,),
                                  start_index_map=(0,))


def _perm16(x, idx):
    return lax.gather(x, idx[:, None], _GDN, slice_sizes=(1,),
                      mode=lax.GatherScatterMode.PROMISE_IN_BOUNDS)


def _hsum_all(x):
    lane = lax.iota(_i32, L)
    for step in (8, 4, 2, 1):
        x = x + _perm16(x, lane ^ step)
    return x


def _zero_vec(ref, n):
    def body(i, _):
        ref[pl.ds(i * L, L)] = jnp.zeros((L,), _f32)
        return 0
    lax.fori_loop(0, n // L, body, 0)


def _zero_rows(ref, rows):
    def body(i, _):
        for j in range(EMB // L):
            ref[i, pl.ds(j * L, L)] = jnp.zeros((L,), _f32)
        return 0
    lax.fori_loop(0, rows, body, 0)


def _gat_scores_body(el_h, er_h, a_h, src_h, dst_h, ee_h, zp_h,
                     idxs_v, idxd_v, ee_v, Lb, Rb, a_v, z_sh, sem1, sem2):
    c = lax.axis_index("c")
    s = lax.axis_index("s")
    wid = s * NC + c

    _zero_vec(ee_v, C)
    for off in (0, 400, 800):
        pltpu.sync_copy(ee_v, z_sh.at[pl.ds(s * ZSTRIPE + off, 400)])
    pltpu.sync_copy(ee_v.at[pl.ds(0, 80)], z_sh.at[pl.ds(s * ZSTRIPE + 1200, 80)])
    plsc.subcore_barrier()

    pltpu.sync_copy(a_h, a_v)

    def chunk(i, _):
        base = wid * PT_ALL + i * C
        pltpu.sync_copy(src_h.at[pl.ds(base, C)], idxs_v)
        pltpu.sync_copy(dst_h.at[pl.ds(base, C)], idxd_v)
        cp1 = pltpu.async_copy(el_h.at[idxs_v], Lb, sem1)
        cp2 = pltpu.async_copy(er_h.at[idxd_v], Rb, sem2)
        cp1.wait()
        cp2.wait()

        def group(g, _):
            a_vecs = [a_v[pl.ds(j * L, L)] for j in range(EMB // L)]
            lane = jax.lax.iota(_i32, L)
            sc16 = jnp.zeros((L,), _f32)
            for k in range(L):
                row = g * L + k
                acc_l = jnp.zeros((L,), _f32)
                acc_a = jnp.zeros((L,), _f32)
                for j in range(EMB // L):
                    sv = Lb[row, pl.ds(j * L, L)] + Rb[row, pl.ds(j * L, L)]
                    acc_l = acc_l + a_vecs[j] * sv
                    acc_a = acc_a + a_vecs[j] * jnp.abs(sv)
                sk = _hsum_all(0.6 * acc_l + 0.4 * acc_a)
                sc16 = jnp.where(lane == k, sk, sc16)
            ee_v[pl.ds(g * L, L)] = jnp.exp(sc16)
            return 0

        lax.fori_loop(0, GRP, group, 0)
        pltpu.sync_copy(ee_v, ee_h.at[pl.ds(base, C)])
        pltpu.sync_copy(ee_v, z_sh.at[idxd_v], add=True)
        return 0

    lax.fori_loop(0, PT_ALL // C, chunk, 0)
    plsc.subcore_barrier()
    pltpu.sync_copy(z_sh.at[pl.ds(s * ZSTRIPE, ZSTRIPE)],
                    zp_h.at[pl.ds(c * NPAD + s * ZSTRIPE, ZSTRIPE)])


def _sc_gat_scores(el, er, a, src, dst):
    fn = pl.kernel(
        _gat_scores_body,
        out_type=(jax.ShapeDtypeStruct((E,), _f32),
                  jax.ShapeDtypeStruct((NC * NPAD,), _f32)),
        scratch_types=[
            pltpu.VMEM((C,), _i32),
            pltpu.VMEM((C,), _i32),
            pltpu.VMEM((C,), _f32),
            pltpu.VMEM((C, EMB), _f32),
            pltpu.VMEM((C, EMB), _f32),
            pltpu.VMEM((EMB,), _f32),
            pltpu.VMEM_SHARED((NPAD,), _f32),
            pltpu.SemaphoreType.DMA,
            pltpu.SemaphoreType.DMA,
        ],
        mesh=_mesh,
    )
    return fn(el, er, a, src, dst)


def _deg_body(dst_h, zp_h, idxd_v, ones_v, z_sh, sem1):
    c = lax.axis_index("c")
    s = lax.axis_index("s")
    wid = s * NC + c

    _zero_vec(ones_v, C)
    for off in (0, 400, 800):
        pltpu.sync_copy(ones_v, z_sh.at[pl.ds(s * ZSTRIPE + off, 400)])
    pltpu.sync_copy(ones_v.at[pl.ds(0, 80)], z_sh.at[pl.ds(s * ZSTRIPE + 1200, 80)])
    plsc.subcore_barrier()

    def fill(i, _):
        ones_v[pl.ds(i * L, L)] = jnp.ones((L,), _f32)
        return 0
    lax.fori_loop(0, C // L, fill, 0)

    def chunk(i, _):
        base = wid * PT_ALL + i * C
        pltpu.sync_copy(dst_h.at[pl.ds(base, C)], idxd_v)
        pltpu.sync_copy(ones_v, z_sh.at[idxd_v], add=True)
        return 0

    lax.fori_loop(0, PT_ALL // C, chunk, 0)
    plsc.subcore_barrier()
    pltpu.sync_copy(z_sh.at[pl.ds(s * ZSTRIPE, ZSTRIPE)],
                    zp_h.at[pl.ds(c * NPAD + s * ZSTRIPE, ZSTRIPE)])


def _sc_deg(dst):
    fn = pl.kernel(
        _deg_body,
        out_type=jax.ShapeDtypeStruct((NC * NPAD,), _f32),
        scratch_types=[
            pltpu.VMEM((C,), _i32),
            pltpu.VMEM((C,), _f32),
            pltpu.VMEM_SHARED((NPAD,), _f32),
            pltpu.SemaphoreType.DMA,
        ],
        mesh=_mesh,
    )
    return fn(dst)


def _wscatter_body(weighted, rows_h, w_h, zinv_h, src_h, dst_h, out_h,
                   idxs_v, idxd_v, idxloc_v, w_v, zw_v, Vb, acc_sh, sem1, sem2):
    c = lax.axis_index("c")
    s = lax.axis_index("s")
    half0 = c * HALF

    _zero_rows(Vb, CW)
    for off in (0, 160, 320, 480):
        ln = min(CW, ACC_STRIPE - off)
        pltpu.sync_copy(Vb.at[pl.ds(0, ln)],
                        acc_sh.at[pl.ds(s * ACC_STRIPE + off, ln)])
    plsc.subcore_barrier()

    def chunk(i, _):
        base = s * PT_SC + i * CW
        pltpu.sync_copy(src_h.at[pl.ds(base, CW)], idxs_v)
        pltpu.sync_copy(dst_h.at[pl.ds(base, CW)], idxd_v)
        if weighted:
            pltpu.sync_copy(w_h.at[pl.ds(base, CW)], w_v)
        cp1 = pltpu.async_copy(rows_h.at[idxs_v], Vb, sem1)
        if weighted:
            pltpu.async_copy(zinv_h.at[idxd_v], zw_v, sem2).wait()
        cp1.wait()

        def group(g, _):
            d16 = idxd_v[pl.ds(g * L, L)]
            inhalf = (d16 >= half0) & (d16 < half0 + HALF)
            loc = jnp.where(inhalf, d16 - half0, HALF)
            idxloc_v[pl.ds(g * L, L)] = loc
            if weighted:
                wv = w_v[pl.ds(g * L, L)] * zw_v[pl.ds(g * L, L)]
                for k in range(L):
                    row = g * L + k
                    wk = wv[k]
                    for j in range(EMB // L):
                        Vb[row, pl.ds(j * L, L)] = Vb[row, pl.ds(j * L, L)] * wk
            return 0

        lax.fori_loop(0, GRPW, group, 0)
        pltpu.sync_copy(Vb, acc_sh.at[idxloc_v], add=True)
        return 0

    lax.fori_loop(0, PT_SC // CW, chunk, 0)
    plsc.subcore_barrier()

    @pl.when(s < NS - 1)
    def _():
        pltpu.sync_copy(acc_sh.at[pl.ds(s * ACC_STRIPE, ACC_STRIPE)],
                        out_h.at[pl.ds(c * HALF + s * ACC_STRIPE, ACC_STRIPE)])

    @pl.when(s == NS - 1)
    def _():
        pltpu.sync_copy(acc_sh.at[pl.ds((NS - 1) * ACC_STRIPE, HALF - (NS - 1) * ACC_STRIPE)],
                        out_h.at[pl.ds(c * HALF + (NS - 1) * ACC_STRIPE, HALF - (NS - 1) * ACC_STRIPE)])


def _sc_wscatter(rows, w, zinv, src, dst, weighted):
    fn = pl.kernel(
        functools.partial(_wscatter_body, weighted),
        out_type=jax.ShapeDtypeStruct((N, EMB), _f32),
        scratch_types=[
            pltpu.VMEM((CW,), _i32),
            pltpu.VMEM((CW,), _i32),
            pltpu.VMEM((CW,), _i32),
            pltpu.VMEM((CW,), _f32),
            pltpu.VMEM((CW,), _f32),
            pltpu.VMEM((CW, EMB), _f32),
            pltpu.VMEM_SHARED((ACC_ROWS, EMB), _f32),
            pltpu.SemaphoreType.DMA,
            pltpu.SemaphoreType.DMA,
        ],
        mesh=_mesh,
    )
    return fn(rows, w, zinv, src, dst)


def _edge_dot_body(a_h, b_h, src_h, dst_h, out_h,
                   idxs_v, idxd_v, out_v, Ab, Bb, sem1, sem2):
    c = lax.axis_index("c")
    s = lax.axis_index("s")
    wid = s * NC + c

    def chunk(i, _):
        base = wid * PT_ALL + i * C
        pltpu.sync_copy(src_h.at[pl.ds(base, C)], idxs_v)
        pltpu.sync_copy(dst_h.at[pl.ds(base, C)], idxd_v)
        cp1 = pltpu.async_copy(a_h.at[idxs_v], Ab, sem1)
        cp2 = pltpu.async_copy(b_h.at[idxd_v], Bb, sem2)
        cp1.wait()
        cp2.wait()

        def group(g, _):
            lane = jax.lax.iota(_i32, L)
            dot16 = jnp.zeros((L,), _f32)
            for k in range(L):
                row = g * L + k
                acc = jnp.zeros((L,), _f32)
                for j in range(EMB // L):
                    acc = acc + Ab[row, pl.ds(j * L, L)] * Bb[row, pl.ds(j * L, L)]
                dot16 = jnp.where(lane == k, _hsum_all(acc), dot16)
            out_v[pl.ds(g * L, L)] = dot16
            return 0

        lax.fori_loop(0, GRP, group, 0)
        pltpu.sync_copy(out_v, out_h.at[pl.ds(base, C)])
        return 0

    lax.fori_loop(0, PT_ALL // C, chunk, 0)


def _sc_edge_dot(a, b, src, dst):
    fn = pl.kernel(
        _edge_dot_body,
        out_type=jax.ShapeDtypeStruct((E,), _f32),
        scratch_types=[
            pltpu.VMEM((C,), _i32),
            pltpu.VMEM((C,), _i32),
            pltpu.VMEM((C,), _f32),
            pltpu.VMEM((C, EMB), _f32),
            pltpu.VMEM((C, EMB), _f32),
            pltpu.SemaphoreType.DMA,
            pltpu.SemaphoreType.DMA,
        ],
        mesh=_mesh,
    )
    return fn(a, b, src, dst)


def _lrelu(x, s):
    return jnp.where(x >= 0, x, s * x)


def _batch_norm(x, g, b, eps=1e-5):
    mu = x.mean(0, keepdims=True)
    var = ((x - mu) ** 2).mean(0, keepdims=True)
    return (x - mu) / jnp.sqrt(var + eps) * g + b


def _gatv2(x, src, dst, p):
    el = x @ p['Ws'] + p['bs']
    er = x @ p['Wd'] + p['bd']
    ee, zpart = _sc_gat_scores(el, er, p['a'], src, dst)
    z = zpart[:N] + zpart[NPAD:NPAD + N]
    zinv = 1.0 / (z + 1e-9)
    out = _sc_wscatter(el, ee, zinv, src, dst, weighted=True)
    return _lrelu(out, 0.01)


def _cheb(x, src, dst, W, b, lam, dinv):
    dcol = dinv[:, None]

    zw = jnp.zeros((E,), _f32)
    zn = jnp.zeros((N,), _f32)

    def lhat(h):
        agg = _sc_wscatter(h * dcol, zw, zn, src, dst, weighted=False)
        return (2.0 / lam) * (h - agg * dcol) - h

    X0 = x
    X1 = lhat(X0)
    X2 = 2.0 * lhat(X1) - X0
    out = X0 @ W[0] + X1 @ W[1] + X2 @ W[2]
    return _lrelu(out + b, 0.01)


def _mlp_bn(x, p):
    return _lrelu(_batch_norm(x @ p['W'] + p['b'], p['g'], p['be']), 0.01)


def kernel(params, laplacian_lambda_max, g_edge_index, user2item_edge_index,
           reverse_edge_index, item2user_edge_index, social_edge_index):
    lam = laplacian_lambda_max[0]
    emb = _batch_norm(params['emb'], params['bn_g'], params['bn_b'])
    ui = emb

    u2i = user2item_edge_index
    rev = reverse_edge_index
    i2u_ei = item2user_edge_index
    soc = social_edge_index

    h = _gatv2(ui, u2i[0], u2i[1], params['gat_u2i'])
    ii = _gatv2(h, rev[0], rev[1], params['gat_ii'])
    i2u = _gatv2(ui, i2u_ei[0], i2u_ei[1], params['gat_i2u'])
    mask = jnp.sum(i2u, axis=1) != 0
    soc_emb = jnp.where(mask[:, None], i2u, ui)
    si = _gatv2(soc_emb, soc[0], soc[1], params['gat_si'])
    spatial = _mlp_bn(jnp.concatenate([ii, si], axis=1), params['spatial_out'])

    degpart = _sc_deg(soc[1])
    deg = degpart[:N] + degpart[NPAD:NPAD + N]
    dinv = jnp.where(deg > 0, 1.0 / jnp.sqrt(jnp.maximum(deg, 1e-9)), 0.0)
    hs = _cheb(ui, soc[0], soc[1], params['cheb_W'], params['cheb_b'], lam, dinv)
    hs = _cheb(hs, soc[0], soc[1], params['cheb_W'], params['cheb_b'], lam, dinv)
    social_pref = _gatv2(hs, soc[0], soc[1], params['gat_spec'])

    h_uP = _mlp_bn(jnp.concatenate([spatial, ui], axis=1), params['mut_c'])
    h_uS = _mlp_bn(jnp.concatenate([social_pref, ui], axis=1), params['mut_s'])
    h_m = h_uP * h_uS
    h_mP = jnp.concatenate([h_m * jax.nn.softmax(h_uP, axis=1), h_uP], axis=1)
    h_mS = jnp.concatenate([h_m * jax.nn.softmax(h_uS, axis=1), h_uS], axis=1)
    h_new_P = _mlp_bn(h_mP, params['pred_p'])
    h_new_S = _mlp_bn(h_mS, params['pred_s'])
    new_ft = h_new_P + h_new_S
    raw_ft = _mlp_bn(emb, params['raw'])

    ed = _sc_edge_dot(new_ft, raw_ft, g_edge_index[0], g_edge_index[1])
    return ed.reshape(E, 1)
